# baseline (speedup 1.0000x reference)
import functools

import jax
import jax.numpy as jnp
from jax.experimental import pallas as pl

B, S, D = 1, 2048, 768
E, TOPK = 8, 2
I = 341
IP = 384
BT = 256
EPAD = 128


def _moe_body(x_ref, gw_ref, sw_ref, sd_ref, ew_ref, ed_ref, out_ref):
    xt = x_ref[...]
    logits = jnp.dot(xt, gw_ref[...], preferred_element_type=jnp.float32)
    lane = jax.lax.broadcasted_iota(jnp.int32, (BT, EPAD), 1)
    valid = lane < E
    logits = jnp.where(valid, logits, -1e30)
    m = jnp.max(logits, axis=1, keepdims=True)
    ex = jnp.exp(logits - m)
    ex = jnp.where(valid, ex, 0.0)
    probs = ex / jnp.sum(ex, axis=1, keepdims=True)
    p1 = jnp.max(probs, axis=1, keepdims=True)
    sel1 = jnp.min(jnp.where(probs == p1, lane, E), axis=1, keepdims=True)
    probs2 = jnp.where(lane == sel1, -1.0, probs)
    p2 = jnp.max(probs2, axis=1, keepdims=True)
    sel2 = jnp.min(jnp.where(probs2 == p2, lane, E), axis=1, keepdims=True)
    wsum = p1 + p2 + 1e-8
    wsel = (jnp.where(lane == sel1, p1, 0.0)
            + jnp.where(lane == sel2, p2, 0.0)) / wsum

    xb = xt.astype(jnp.bfloat16)

    def ffn(w_gu, w_d):
        gu = jnp.dot(xb, w_gu, preferred_element_type=jnp.float32)
        g = gu[:, :IP]
        u = gu[:, IP:]
        h = (g * jax.nn.sigmoid(g)) * u
        return jnp.dot(h.astype(jnp.bfloat16), w_d,
                       preferred_element_type=jnp.float32)

    acc = ffn(sw_ref[...], sd_ref[...])
    for e in range(E):
        ye = ffn(ew_ref[e], ed_ref[e])
        acc = acc + wsel[:, e:e + 1] * ye
    out_ref[...] = acc


@jax.jit
def _moe(x, gw, sw, sd, ew, ed):
    grid = (S // BT,)
    return pl.pallas_call(
        _moe_body,
        grid=grid,
        in_specs=[
            pl.BlockSpec((BT, D), lambda i: (i, 0)),
            pl.BlockSpec((D, EPAD), lambda i: (0, 0)),
            pl.BlockSpec((D, 2 * IP), lambda i: (0, 0)),
            pl.BlockSpec((IP, D), lambda i: (0, 0)),
            pl.BlockSpec((E, D, 2 * IP), lambda i: (0, 0, 0)),
            pl.BlockSpec((E, IP, D), lambda i: (0, 0, 0)),
        ],
        out_specs=pl.BlockSpec((BT, D), lambda i: (i, 0)),
        out_shape=jax.ShapeDtypeStruct((S, D), jnp.float32),
    )(x, gw, sw, sd, ew, ed)


def kernel(hidden_states, shared_gate_up_w, shared_down_w, expert_gate_up,
           expert_down, gate_weight):
    b, s, d = hidden_states.shape
    x = hidden_states.reshape(s, d)
    gw = jnp.zeros((d, EPAD), jnp.float32).at[:, :E].set(gate_weight.T)
    sw = jnp.zeros((d, 2 * IP), jnp.bfloat16)
    sw = sw.at[:, :I].set(shared_gate_up_w[:I].T.astype(jnp.bfloat16))
    sw = sw.at[:, IP:IP + I].set(shared_gate_up_w[I:].T.astype(jnp.bfloat16))
    sd = jnp.zeros((IP, d), jnp.bfloat16).at[:I].set(
        shared_down_w.T.astype(jnp.bfloat16))
    ew = jnp.zeros((E, d, 2 * IP), jnp.bfloat16)
    ew = ew.at[:, :, :I].set(expert_gate_up[:, :, :I].astype(jnp.bfloat16))
    ew = ew.at[:, :, IP:IP + I].set(
        expert_gate_up[:, :, I:].astype(jnp.bfloat16))
    ed = jnp.zeros((E, IP, d), jnp.bfloat16).at[:, :I].set(
        expert_down.astype(jnp.bfloat16))
    out = _moe(x, gw, sw, sd, ew, ed)
    return out.reshape(b, s, d)


# speedup vs baseline: 1.7338x; 1.7338x over previous
import functools

import jax
import jax.numpy as jnp
from jax.experimental import pallas as pl

B, S, D = 1, 2048, 768
E, TOPK = 8, 2
I = 341
IP = 384
BT = 512


def _moe_body(x_ref, gw_ref, sw_ref, sd_ref, ew_ref, ed_ref, out_ref):
    xt = x_ref[...]
    logits = jnp.dot(xt, gw_ref[...], preferred_element_type=jnp.float32)
    lane = jax.lax.broadcasted_iota(jnp.int32, (BT, E), 1)
    m = jnp.max(logits, axis=1, keepdims=True)
    ex = jnp.exp(logits - m)
    probs = ex / jnp.sum(ex, axis=1, keepdims=True)
    p1 = jnp.max(probs, axis=1, keepdims=True)
    sel1 = jnp.min(jnp.where(probs == p1, lane, E), axis=1, keepdims=True)
    probs2 = jnp.where(lane == sel1, -1.0, probs)
    p2 = jnp.max(probs2, axis=1, keepdims=True)
    sel2 = jnp.min(jnp.where(probs2 == p2, lane, E), axis=1, keepdims=True)
    wsum = p1 + p2 + 1e-8
    wsel = (jnp.where(lane == sel1, p1, 0.0)
            + jnp.where(lane == sel2, p2, 0.0)) / wsum

    xb = xt.astype(jnp.bfloat16)

    def ffn(w_gu, w_d):
        gu = jnp.dot(xb, w_gu, preferred_element_type=jnp.float32)
        g = gu[:, :IP]
        u = gu[:, IP:]
        h = (g * jax.nn.sigmoid(g)) * u
        return jnp.dot(h.astype(jnp.bfloat16), w_d,
                       preferred_element_type=jnp.float32)

    acc = ffn(sw_ref[...], sd_ref[...])
    for e in range(E):
        ye = ffn(ew_ref[e], ed_ref[e])
        acc = acc + wsel[:, e:e + 1] * ye
    out_ref[...] = acc


@jax.jit
def _moe(x, gw, sw, sd, ew, ed):
    grid = (S // BT,)
    return pl.pallas_call(
        _moe_body,
        grid=grid,
        in_specs=[
            pl.BlockSpec((BT, D), lambda i: (i, 0)),
            pl.BlockSpec((D, E), lambda i: (0, 0)),
            pl.BlockSpec((D, 2 * IP), lambda i: (0, 0)),
            pl.BlockSpec((IP, D), lambda i: (0, 0)),
            pl.BlockSpec((E, D, 2 * IP), lambda i: (0, 0, 0)),
            pl.BlockSpec((E, IP, D), lambda i: (0, 0, 0)),
        ],
        out_specs=pl.BlockSpec((BT, D), lambda i: (i, 0)),
        out_shape=jax.ShapeDtypeStruct((S, D), jnp.float32),
    )(x, gw, sw, sd, ew, ed)


def kernel(hidden_states, shared_gate_up_w, shared_down_w, expert_gate_up,
           expert_down, gate_weight):
    b, s, d = hidden_states.shape
    x = hidden_states.reshape(s, d)
    gw = gate_weight.T
    sgu = shared_gate_up_w.astype(jnp.bfloat16)
    zs = jnp.zeros((d, IP - I), jnp.bfloat16)
    sw = jnp.concatenate([sgu[:I].T, zs, sgu[I:].T, zs], axis=1)
    sd = jnp.pad(shared_down_w.T.astype(jnp.bfloat16), ((0, IP - I), (0, 0)))
    egu = expert_gate_up.astype(jnp.bfloat16)
    ze = jnp.zeros((E, d, IP - I), jnp.bfloat16)
    ew = jnp.concatenate([egu[:, :, :I], ze, egu[:, :, I:], ze], axis=2)
    ed = jnp.pad(expert_down.astype(jnp.bfloat16),
                 ((0, 0), (0, IP - I), (0, 0)))
    out = _moe(x, gw, sw, sd, ew, ed)
    return out.reshape(b, s, d)


# speedup vs baseline: 1.7854x; 1.0298x over previous
import functools

import jax
import jax.numpy as jnp
from jax.experimental import pallas as pl

B, S, D = 1, 2048, 768
E, TOPK = 8, 2
I = 341
IP = 384
BT = 1024


def _moe_body(x_ref, gw_ref, sw_ref, sd_ref, ew_ref, ed_ref, out_ref):
    xt = x_ref[...]
    logits = jnp.dot(xt, gw_ref[...], preferred_element_type=jnp.float32)
    lane = jax.lax.broadcasted_iota(jnp.int32, (BT, E), 1)
    m = jnp.max(logits, axis=1, keepdims=True)
    ex = jnp.exp(logits - m)
    probs = ex / jnp.sum(ex, axis=1, keepdims=True)
    p1 = jnp.max(probs, axis=1, keepdims=True)
    sel1 = jnp.min(jnp.where(probs == p1, lane, E), axis=1, keepdims=True)
    probs2 = jnp.where(lane == sel1, -1.0, probs)
    p2 = jnp.max(probs2, axis=1, keepdims=True)
    sel2 = jnp.min(jnp.where(probs2 == p2, lane, E), axis=1, keepdims=True)
    wsum = p1 + p2 + 1e-8
    wsel = (jnp.where(lane == sel1, p1, 0.0)
            + jnp.where(lane == sel2, p2, 0.0)) / wsum

    xb = xt.astype(jnp.bfloat16)

    def ffn(w_gu, w_d):
        gu = jnp.dot(xb, w_gu, preferred_element_type=jnp.float32)
        g = gu[:, :IP]
        u = gu[:, IP:]
        h = (g * jax.nn.sigmoid(g)) * u
        return jnp.dot(h.astype(jnp.bfloat16), w_d,
                       preferred_element_type=jnp.float32)

    acc = ffn(sw_ref[...], sd_ref[...])
    for e in range(E):
        ye = ffn(ew_ref[e], ed_ref[e])
        acc = acc + wsel[:, e:e + 1] * ye
    out_ref[...] = acc


@jax.jit
def _moe(x, gw, sw, sd, ew, ed):
    grid = (S // BT,)
    return pl.pallas_call(
        _moe_body,
        grid=grid,
        in_specs=[
            pl.BlockSpec((BT, D), lambda i: (i, 0)),
            pl.BlockSpec((D, E), lambda i: (0, 0)),
            pl.BlockSpec((D, 2 * IP), lambda i: (0, 0)),
            pl.BlockSpec((IP, D), lambda i: (0, 0)),
            pl.BlockSpec((E, D, 2 * IP), lambda i: (0, 0, 0)),
            pl.BlockSpec((E, IP, D), lambda i: (0, 0, 0)),
        ],
        out_specs=pl.BlockSpec((BT, D), lambda i: (i, 0)),
        out_shape=jax.ShapeDtypeStruct((S, D), jnp.float32),
    )(x, gw, sw, sd, ew, ed)


def kernel(hidden_states, shared_gate_up_w, shared_down_w, expert_gate_up,
           expert_down, gate_weight):
    b, s, d = hidden_states.shape
    x = hidden_states.reshape(s, d)
    gw = gate_weight.T
    sgu = shared_gate_up_w.astype(jnp.bfloat16)
    zs = jnp.zeros((d, IP - I), jnp.bfloat16)
    sw = jnp.concatenate([sgu[:I].T, zs, sgu[I:].T, zs], axis=1)
    sd = jnp.pad(shared_down_w.T.astype(jnp.bfloat16), ((0, IP - I), (0, 0)))
    egu = expert_gate_up.astype(jnp.bfloat16)
    ze = jnp.zeros((E, d, IP - I), jnp.bfloat16)
    ew = jnp.concatenate([egu[:, :, :I], ze, egu[:, :, I:], ze], axis=2)
    ed = jnp.pad(expert_down.astype(jnp.bfloat16),
                 ((0, 0), (0, IP - I), (0, 0)))
    out = _moe(x, gw, sw, sd, ew, ed)
    return out.reshape(b, s, d)


# speedup vs baseline: 1.9694x; 1.1031x over previous
import jax
import jax.numpy as jnp
from jax.experimental import pallas as pl

B, S, D = 1, 2048, 768
E, TOPK = 8, 2
I = 341
BT = 1024


def _moe_body(x_ref, gw_ref, swg_ref, swu_ref, sd_ref, ewg_ref, ewu_ref,
              ed_ref, out_ref):
    xt = x_ref[...]
    logits = jnp.dot(xt, gw_ref[...], preferred_element_type=jnp.float32)
    lane = jax.lax.broadcasted_iota(jnp.int32, (BT, E), 1)
    m = jnp.max(logits, axis=1, keepdims=True)
    ex = jnp.exp(logits - m)
    probs = ex / jnp.sum(ex, axis=1, keepdims=True)
    p1 = jnp.max(probs, axis=1, keepdims=True)
    sel1 = jnp.min(jnp.where(probs == p1, lane, E), axis=1, keepdims=True)
    probs2 = jnp.where(lane == sel1, -1.0, probs)
    p2 = jnp.max(probs2, axis=1, keepdims=True)
    sel2 = jnp.min(jnp.where(probs2 == p2, lane, E), axis=1, keepdims=True)
    wsum = p1 + p2 + 1e-8
    wsel = (jnp.where(lane == sel1, p1, 0.0)
            + jnp.where(lane == sel2, p2, 0.0)) / wsum

    xb = xt.astype(jnp.bfloat16)

    def ffn(w_g, w_u, w_d):
        g = jnp.dot(xb, w_g, preferred_element_type=jnp.float32)
        u = jnp.dot(xb, w_u, preferred_element_type=jnp.float32)
        h = (g * jax.nn.sigmoid(g)) * u
        return jnp.dot(h.astype(jnp.bfloat16), w_d,
                       preferred_element_type=jnp.float32)

    acc = ffn(swg_ref[...], swu_ref[...], sd_ref[...])
    for e in range(E):
        ye = ffn(ewg_ref[e], ewu_ref[e], ed_ref[e])
        acc = acc + wsel[:, e:e + 1] * ye
    out_ref[...] = acc


@jax.jit
def _moe(x, gw, swg, swu, sd, ewg, ewu, ed):
    grid = (S // BT,)
    return pl.pallas_call(
        _moe_body,
        grid=grid,
        in_specs=[
            pl.BlockSpec((BT, D), lambda i: (i, 0)),
            pl.BlockSpec((D, E), lambda i: (0, 0)),
            pl.BlockSpec((D, I), lambda i: (0, 0)),
            pl.BlockSpec((D, I), lambda i: (0, 0)),
            pl.BlockSpec((I, D), lambda i: (0, 0)),
            pl.BlockSpec((E, D, I), lambda i: (0, 0, 0)),
            pl.BlockSpec((E, D, I), lambda i: (0, 0, 0)),
            pl.BlockSpec((E, I, D), lambda i: (0, 0, 0)),
        ],
        out_specs=pl.BlockSpec((BT, D), lambda i: (i, 0)),
        out_shape=jax.ShapeDtypeStruct((S, D), jnp.float32),
    )(x, gw, swg, swu, sd, ewg, ewu, ed)


def kernel(hidden_states, shared_gate_up_w, shared_down_w, expert_gate_up,
           expert_down, gate_weight):
    b, s, d = hidden_states.shape
    x = hidden_states.reshape(s, d)
    gw = gate_weight.T
    sgu = shared_gate_up_w.astype(jnp.bfloat16)
    swg = sgu[:I].T
    swu = sgu[I:].T
    sd = shared_down_w.T.astype(jnp.bfloat16)
    egu = expert_gate_up.astype(jnp.bfloat16)
    ewg = egu[:, :, :I]
    ewu = egu[:, :, I:]
    ed = expert_down.astype(jnp.bfloat16)
    out = _moe(x, gw, swg, swu, sd, ewg, ewu, ed)
    return out.reshape(b, s, d)


# speedup vs baseline: 1.9764x; 1.0035x over previous
import jax
import jax.numpy as jnp
from jax.experimental import pallas as pl

B, S, D = 1, 2048, 768
E, TOPK = 8, 2
I = 341
BT = 1024


def _moe_body(x_ref, gw_ref, sw_ref, sd_ref, ew_ref, ed_ref, out_ref):
    xt = x_ref[...]
    logits = jnp.dot(xt, gw_ref[...], preferred_element_type=jnp.float32)
    lane = jax.lax.broadcasted_iota(jnp.int32, (BT, E), 1)
    m = jnp.max(logits, axis=1, keepdims=True)
    ex = jnp.exp(logits - m)
    probs = ex / jnp.sum(ex, axis=1, keepdims=True)
    p1 = jnp.max(probs, axis=1, keepdims=True)
    sel1 = jnp.min(jnp.where(probs == p1, lane, E), axis=1, keepdims=True)
    probs2 = jnp.where(lane == sel1, -1.0, probs)
    p2 = jnp.max(probs2, axis=1, keepdims=True)
    sel2 = jnp.min(jnp.where(probs2 == p2, lane, E), axis=1, keepdims=True)
    wsum = p1 + p2 + 1e-8
    wsel = (jnp.where(lane == sel1, p1, 0.0)
            + jnp.where(lane == sel2, p2, 0.0)) / wsum

    xb = xt.astype(jnp.bfloat16)

    def ffn(w_gu, w_d):
        gu = jnp.dot(xb, w_gu, preferred_element_type=jnp.float32)
        g = gu[:, :I]
        u = gu[:, I:]
        h = (g * jax.nn.sigmoid(g)) * u
        return jnp.dot(h.astype(jnp.bfloat16), w_d,
                       preferred_element_type=jnp.float32)

    acc = ffn(sw_ref[...], sd_ref[...])
    for e in range(E):
        ye = ffn(ew_ref[e], ed_ref[e])
        acc = acc + wsel[:, e:e + 1] * ye
    out_ref[...] = acc


@jax.jit
def _moe(x, gw, sw, sd, ew, ed):
    grid = (S // BT,)
    return pl.pallas_call(
        _moe_body,
        grid=grid,
        in_specs=[
            pl.BlockSpec((BT, D), lambda i: (i, 0)),
            pl.BlockSpec((D, E), lambda i: (0, 0)),
            pl.BlockSpec((D, 2 * I), lambda i: (0, 0)),
            pl.BlockSpec((I, D), lambda i: (0, 0)),
            pl.BlockSpec((E, D, 2 * I), lambda i: (0, 0, 0)),
            pl.BlockSpec((E, I, D), lambda i: (0, 0, 0)),
        ],
        out_specs=pl.BlockSpec((BT, D), lambda i: (i, 0)),
        out_shape=jax.ShapeDtypeStruct((S, D), jnp.float32),
    )(x, gw, sw, sd, ew, ed)


def kernel(hidden_states, shared_gate_up_w, shared_down_w, expert_gate_up,
           expert_down, gate_weight):
    b, s, d = hidden_states.shape
    x = hidden_states.reshape(s, d)
    gw = gate_weight.T
    sw = shared_gate_up_w.T.astype(jnp.bfloat16)
    sd = shared_down_w.T.astype(jnp.bfloat16)
    ew = expert_gate_up.astype(jnp.bfloat16)
    ed = expert_down.astype(jnp.bfloat16)
    out = _moe(x, gw, sw, sd, ew, ed)
    return out.reshape(b, s, d)
